# baseline (speedup 1.0000x reference)
import functools

import jax
import jax.numpy as jnp
from jax import lax
from jax.experimental import pallas as pl
from jax.experimental.pallas import tpu as pltpu
from jax.experimental.pallas import tpu_sc as plsc

_NC = 2
_NS = 16
_NW = _NC * _NS
_BB = 128


@functools.partial(jax.jit, static_argnames=("B", "S", "D"))
def _sc_gather(tokens_t, wte, *, B, S, D):
    assert B == _NW * _BB

    mesh = plsc.VectorSubcoreMesh(
        core_axis_name="c", subcore_axis_name="s",
        num_cores=_NC, num_subcores=_NS)

    @functools.partial(
        pl.kernel,
        out_type=jax.ShapeDtypeStruct((S, D // 8, B // _BB, 8, _BB),
                                      jnp.float32),
        mesh=mesh,
        scratch_types=[
            pltpu.VMEM((S, _BB), jnp.int32),
            pltpu.VMEM((_BB, D), jnp.float32),
            pltpu.VMEM((_BB, D), jnp.float32),
            pltpu.VMEM((D, _BB), jnp.float32),
            pltpu.VMEM((D, _BB), jnp.float32),
            pltpu.SemaphoreType.DMA,
            pltpu.SemaphoreType.DMA,
            pltpu.SemaphoreType.DMA,
            pltpu.SemaphoreType.DMA,
        ],
        compiler_params=pltpu.CompilerParams(
            use_tc_tiling_on_sc=False, needs_layout_passes=False),
    )
    def k(tok_hbm, wte_hbm, out_hbm,
          idx_all, rows_v0, rows_v1, t_v0, t_v1,
          gsem0, gsem1, osem0, osem1):
        rows_b = (rows_v0, rows_v1)
        t_b = (t_v0, t_v1)
        gsem = (gsem0, gsem1)
        osem = (osem0, osem1)
        wid = lax.axis_index("s") * _NC + lax.axis_index("c")

        pltpu.sync_copy(tok_hbm.at[:, pl.ds(wid * _BB, _BB)], idx_all)

        iotas = [lax.iota(jnp.int32, 16) + d0 for d0 in range(0, D, 16)]

        def gather(s, p):
            return pltpu.async_copy(
                wte_hbm.at[idx_all.at[s]], rows_b[p], gsem[p])

        def out_writes(s, p):
            for dh in range(D // 8):
                pltpu.async_copy(
                    t_b[p].at[pl.ds(dh * 8, 8), :], out_hbm.at[s, dh, wid],
                    osem[p])

        def out_drain(s, p):
            for dh in range(D // 8):
                pltpu.make_async_copy(
                    t_b[p].at[pl.ds(dh * 8, 8), :], out_hbm.at[s, dh, wid],
                    osem[p]).wait()

        gather(0, 0)

        @pl.loop(0, S, step=2)
        def s_loop(s2):
            for q in range(2):
                s = s2 + q
                p = q

                @pl.when(s + 1 < S)
                def _():
                    gather(s + 1, 1 - p)

                pltpu.make_async_copy(
                    wte_hbm.at[idx_all.at[s]], rows_b[p], gsem[p]).wait()

                @pl.when(s >= 2)
                def _():
                    out_drain(s - 2, p)

                for g0 in range(0, _BB, 8):
                    vals = [rows_b[p][bl, pl.ds(v * 16, 16)]
                            for bl in range(g0, g0 + 8)
                            for v in range(D // 16)]
                    for i, bl in enumerate(range(g0, g0 + 8)):
                        col = jnp.full((16,), bl, jnp.int32)
                        for v in range(D // 16):
                            plsc.store_scatter(
                                t_b[p], [iotas[v], col],
                                vals[i * (D // 16) + v])

                out_writes(s, p)

        for s in (S - 2, S - 1):
            out_drain(s, s % 2)

    return k(tokens_t, wte)


def kernel(tokens, wte, avg, var):
    B, S = tokens.shape
    _, D = wte.shape
    NT = avg.shape[0]
    tokens_t = jnp.swapaxes(tokens.astype(jnp.int32), 0, 1)
    out5 = _sc_gather(tokens_t, wte, B=B, S=S, D=D)
    sample = jax.random.normal(jax.random.key(42), (B, NT, D), dtype=wte.dtype)
    prefix = sample * var[None, :, :] + avg[None, :, :]
    out = out5.transpose(2, 4, 0, 1, 3).reshape(B, S, D)
    return lax.dynamic_update_slice(out, prefix.astype(out.dtype), (0, 0, 0))
